# baseline (device time: 52521 ns/iter reference)
import jax
import jax.numpy as jnp
from jax import lax
from jax.experimental import pallas as pl
from jax.experimental.pallas import tpu as pltpu

T = 2048
D = 1024
V_SHARD = 16384
HALF = T // 2
C = 8
R = HALF // C
G = 8


def kernel(ids, E):
    ids1d = ids.astype(jnp.int32)
    ids2d = ids1d.reshape(T, 1)

    my_x = lax.axis_index("x")
    my_y = lax.axis_index("y")

    seg_ids = lax.dynamic_slice(ids1d, (my_y * HALF,), (HALF,))
    idx_loc = jnp.clip(seg_ids - my_x * V_SHARD, 0, V_SHARD - 1)

    def body(
        idx_smem,
        ids_vmem,
        E_hbm,
        out_ref,
        gbuf_ref,
        part_ref,
        xrecv_ref,
        gsems,
        x_send_sems,
        x_recv_sems,
        y_send_sems,
        y_recv_sems,
    ):
        x = lax.axis_index("x")
        y = lax.axis_index("y")
        xnbr = (1 - x, y)
        ynbr = (x, 1 - y)

        barrier = pltpu.get_barrier_semaphore()
        for nbr in (xnbr, ynbr):
            pl.semaphore_signal(
                barrier, inc=1, device_id=nbr, device_id_type=pl.DeviceIdType.MESH
            )
        pl.semaphore_wait(barrier, 2)

        tok0 = y * HALF

        x_rdmas = []
        y_rdmas = []
        for c in range(C):
            rows = pl.ds(c * R, R)
            tok_rows = pl.ds(tok0 + c * R, R)
            x_rdmas.append(
                pltpu.make_async_remote_copy(
                    src_ref=part_ref.at[rows],
                    dst_ref=xrecv_ref.at[rows],
                    send_sem=x_send_sems.at[c],
                    recv_sem=x_recv_sems.at[c],
                    device_id=xnbr,
                    device_id_type=pl.DeviceIdType.MESH,
                )
            )
            y_rdmas.append(
                pltpu.make_async_remote_copy(
                    src_ref=out_ref.at[tok_rows],
                    dst_ref=out_ref.at[tok_rows],
                    send_sem=y_send_sems.at[c],
                    recv_sem=y_recv_sems.at[c],
                    device_id=ynbr,
                    device_id_type=pl.DeviceIdType.MESH,
                )
            )

        def issue_chunk(c):
            def body8(k, _):
                off = c * R + k * G
                for u in range(G):
                    pltpu.make_async_copy(
                        E_hbm.at[pl.ds(idx_smem[off + u], 1), :],
                        gbuf_ref.at[pl.ds(off + u, 1), :],
                        gsems.at[c],
                    ).start()
                return 0

            lax.fori_loop(0, R // G, body8, 0, unroll=4)

        def flush_chunk(c):
            def wait8(k, _):
                for _u in range(G):
                    pltpu.make_async_copy(
                        E_hbm.at[pl.ds(0, 1), :],
                        gbuf_ref.at[pl.ds(0, 1), :],
                        gsems.at[c],
                    ).wait()
                return 0

            lax.fori_loop(0, R // G, wait8, 0, unroll=4)
            rows = pl.ds(c * R, R)
            part_ref[rows] = gbuf_ref[rows].astype(jnp.bfloat16)
            x_rdmas[c].start()

        issue_chunk(0)
        for c in range(1, C):
            issue_chunk(c)
            flush_chunk(c - 1)
        flush_chunk(C - 1)

        for c in range(C):
            x_rdmas[c].wait_recv()
            rows = pl.ds(c * R, R)
            tok_rows = pl.ds(tok0 + c * R, R)
            mine = (ids_vmem[tok_rows] // V_SHARD) == x
            out_ref[tok_rows] = jnp.where(mine, part_ref[rows], xrecv_ref[rows])
            y_rdmas[c].start()

        for c in range(C):
            y_rdmas[c].wait_recv()

        for c in range(C):
            x_rdmas[c].wait_send()
            y_rdmas[c].wait_send()

    return pl.pallas_call(
        body,
        out_shape=jax.ShapeDtypeStruct((T, D), jnp.bfloat16),
        in_specs=[
            pl.BlockSpec(memory_space=pltpu.SMEM),
            pl.BlockSpec(memory_space=pltpu.VMEM),
            pl.BlockSpec(memory_space=pltpu.HBM),
        ],
        out_specs=pl.BlockSpec(memory_space=pltpu.VMEM),
        scratch_shapes=[
            pltpu.VMEM((HALF, D), jnp.float32),
            pltpu.VMEM((HALF, D), jnp.bfloat16),
            pltpu.VMEM((HALF, D), jnp.bfloat16),
            pltpu.SemaphoreType.DMA((C,)),
            pltpu.SemaphoreType.DMA((C,)),
            pltpu.SemaphoreType.DMA((C,)),
            pltpu.SemaphoreType.DMA((C,)),
            pltpu.SemaphoreType.DMA((C,)),
        ],
        compiler_params=pltpu.CompilerParams(collective_id=0),
    )(idx_loc, ids2d, E)


# device time: 50104 ns/iter; 1.0482x vs baseline; 1.0482x over previous
import jax
import jax.numpy as jnp
from jax import lax
from jax.experimental import pallas as pl
from jax.experimental.pallas import tpu as pltpu

T = 2048
D = 1024
V_SHARD = 16384
HALF = T // 2
C = 8
R = HALF // C
G = 8


def kernel(ids, E):
    ids1d = ids.astype(jnp.int32)
    ids2d = ids1d.reshape(T, 1)

    my_x = lax.axis_index("x")
    my_y = lax.axis_index("y")

    seg_ids = lax.dynamic_slice(ids1d, (my_y * HALF,), (HALF,))
    idx_loc = seg_ids - my_x * V_SHARD
    owned = (idx_loc >= 0) & (idx_loc < V_SHARD)
    counts = owned.reshape(C, R).sum(axis=1).astype(jnp.int32)

    def body(
        idx_smem,
        counts_smem,
        ids_vmem,
        E_hbm,
        out_ref,
        gbuf_ref,
        part_ref,
        xrecv_ref,
        gsems,
        x_send_sems,
        x_recv_sems,
        y_send_sems,
        y_recv_sems,
    ):
        x = lax.axis_index("x")
        y = lax.axis_index("y")
        xnbr = (1 - x, y)
        ynbr = (x, 1 - y)

        barrier = pltpu.get_barrier_semaphore()
        for nbr in (xnbr, ynbr):
            pl.semaphore_signal(
                barrier, inc=1, device_id=nbr, device_id_type=pl.DeviceIdType.MESH
            )

        tok0 = y * HALF

        x_rdmas = []
        y_rdmas = []
        for c in range(C):
            rows = pl.ds(c * R, R)
            tok_rows = pl.ds(tok0 + c * R, R)
            x_rdmas.append(
                pltpu.make_async_remote_copy(
                    src_ref=part_ref.at[rows],
                    dst_ref=xrecv_ref.at[rows],
                    send_sem=x_send_sems.at[c],
                    recv_sem=x_recv_sems.at[c],
                    device_id=xnbr,
                    device_id_type=pl.DeviceIdType.MESH,
                )
            )
            y_rdmas.append(
                pltpu.make_async_remote_copy(
                    src_ref=out_ref.at[tok_rows],
                    dst_ref=out_ref.at[tok_rows],
                    send_sem=y_send_sems.at[c],
                    recv_sem=y_recv_sems.at[c],
                    device_id=ynbr,
                    device_id_type=pl.DeviceIdType.MESH,
                )
            )

        def issue_chunk(c):
            def body8(k, _):
                off = c * R + k * G
                for u in range(G):
                    idx = idx_smem[off + u]

                    @pl.when(jnp.logical_and(idx >= 0, idx < V_SHARD))
                    def _():
                        pltpu.make_async_copy(
                            E_hbm.at[pl.ds(idx, 1), :],
                            gbuf_ref.at[pl.ds(off + u, 1), :],
                            gsems.at[c],
                        ).start()

                return 0

            lax.fori_loop(0, R // G, body8, 0, unroll=4)

        def flush_chunk(c):
            def wait1(k, _):
                pltpu.make_async_copy(
                    E_hbm.at[pl.ds(0, 1), :],
                    gbuf_ref.at[pl.ds(0, 1), :],
                    gsems.at[c],
                ).wait()
                return 0

            lax.fori_loop(0, counts_smem[c], wait1, 0)
            rows = pl.ds(c * R, R)
            part_ref[rows] = gbuf_ref[rows].astype(jnp.bfloat16)
            x_rdmas[c].start()

        issue_chunk(0)
        pl.semaphore_wait(barrier, 2)
        for c in range(1, C):
            issue_chunk(c)
            flush_chunk(c - 1)
        flush_chunk(C - 1)

        for c in range(C):
            x_rdmas[c].wait_recv()
            rows = pl.ds(c * R, R)
            tok_rows = pl.ds(tok0 + c * R, R)
            mine = (ids_vmem[tok_rows] // V_SHARD) == x
            out_ref[tok_rows] = jnp.where(mine, part_ref[rows], xrecv_ref[rows])
            y_rdmas[c].start()

        for c in range(C):
            y_rdmas[c].wait_recv()

        for c in range(C):
            x_rdmas[c].wait_send()
            y_rdmas[c].wait_send()

    return pl.pallas_call(
        body,
        out_shape=jax.ShapeDtypeStruct((T, D), jnp.bfloat16),
        in_specs=[
            pl.BlockSpec(memory_space=pltpu.SMEM),
            pl.BlockSpec(memory_space=pltpu.SMEM),
            pl.BlockSpec(memory_space=pltpu.VMEM),
            pl.BlockSpec(memory_space=pltpu.HBM),
        ],
        out_specs=pl.BlockSpec(memory_space=pltpu.VMEM),
        scratch_shapes=[
            pltpu.VMEM((HALF, D), jnp.float32),
            pltpu.VMEM((HALF, D), jnp.bfloat16),
            pltpu.VMEM((HALF, D), jnp.bfloat16),
            pltpu.SemaphoreType.DMA((C,)),
            pltpu.SemaphoreType.DMA((C,)),
            pltpu.SemaphoreType.DMA((C,)),
            pltpu.SemaphoreType.DMA((C,)),
            pltpu.SemaphoreType.DMA((C,)),
        ],
        compiler_params=pltpu.CompilerParams(collective_id=0),
    )(idx_loc, counts, ids2d, E)


# device time: 44638 ns/iter; 1.1766x vs baseline; 1.1225x over previous
import jax
import jax.numpy as jnp
from jax import lax
from jax.experimental import pallas as pl
from jax.experimental.pallas import tpu as pltpu

T = 2048
D = 1024
V_SHARD = 16384
HALF = T // 2
C = 8
R = HALF // C
G = 8
SEG = R + G


def kernel(ids, E):
    ids1d = ids.astype(jnp.int32)
    ids2d = ids1d.reshape(T, 1)

    my_x = lax.axis_index("x")
    my_y = lax.axis_index("y")

    seg_ids = lax.dynamic_slice(ids1d, (my_y * HALF,), (HALF,))
    idx_loc = seg_ids - my_x * V_SHARD

    def body(
        idx_smem,
        ids_vmem,
        E_hbm,
        out_ref,
        gbuf_ref,
        part_ref,
        xrecv_ref,
        src_l,
        dst_l,
        gsems,
        x_send_sems,
        x_recv_sems,
        y_send_sems,
        y_recv_sems,
    ):
        x = lax.axis_index("x")
        y = lax.axis_index("y")
        xnbr = (1 - x, y)
        ynbr = (x, 1 - y)

        barrier = pltpu.get_barrier_semaphore()
        for nbr in (xnbr, ynbr):
            pl.semaphore_signal(
                barrier, inc=1, device_id=nbr, device_id_type=pl.DeviceIdType.MESH
            )

        tok0 = y * HALF

        x_rdmas = []
        y_rdmas = []
        for c in range(C):
            rows = pl.ds(c * R, R)
            tok_rows = pl.ds(tok0 + c * R, R)
            x_rdmas.append(
                pltpu.make_async_remote_copy(
                    src_ref=part_ref.at[rows],
                    dst_ref=xrecv_ref.at[rows],
                    send_sem=x_send_sems.at[c],
                    recv_sem=x_recv_sems.at[c],
                    device_id=xnbr,
                    device_id_type=pl.DeviceIdType.MESH,
                )
            )
            y_rdmas.append(
                pltpu.make_async_remote_copy(
                    src_ref=out_ref.at[tok_rows],
                    dst_ref=out_ref.at[tok_rows],
                    send_sem=y_send_sems.at[c],
                    recv_sem=y_recv_sems.at[c],
                    device_id=ynbr,
                    device_id_type=pl.DeviceIdType.MESH,
                )
            )

        def scan_issue_chunk(c):
            base = c * SEG

            def scan_row(t, cnt):
                v = idx_smem[c * R + t]
                ok = jnp.logical_and(v >= 0, v < V_SHARD)
                src_l[base + cnt] = v
                dst_l[base + cnt] = c * R + t
                return cnt + ok.astype(jnp.int32)

            cnt = lax.fori_loop(0, R, scan_row, 0, unroll=4)
            for u in range(G):
                src_l[base + cnt + u] = 0
                dst_l[base + cnt + u] = HALF
            n8 = (cnt + G - 1) // G

            def body8(k, _):
                off = base + k * G
                for u in range(G):
                    pltpu.make_async_copy(
                        E_hbm.at[pl.ds(src_l[off + u], 1), :],
                        gbuf_ref.at[pl.ds(dst_l[off + u], 1), :],
                        gsems.at[c],
                    ).start()
                return 0

            lax.fori_loop(0, n8, body8, 0)
            return n8

        def flush_chunk(c, n8):
            def wait8(k, _):
                for _u in range(G):
                    pltpu.make_async_copy(
                        E_hbm.at[pl.ds(0, 1), :],
                        gbuf_ref.at[pl.ds(0, 1), :],
                        gsems.at[c],
                    ).wait()
                return 0

            lax.fori_loop(0, n8, wait8, 0)
            rows = pl.ds(c * R, R)
            part_ref[rows] = gbuf_ref[rows].astype(jnp.bfloat16)
            x_rdmas[c].start()

        n8s = [None] * C
        n8s[0] = scan_issue_chunk(0)
        pl.semaphore_wait(barrier, 2)
        for c in range(1, C):
            n8s[c] = scan_issue_chunk(c)
            flush_chunk(c - 1, n8s[c - 1])
        flush_chunk(C - 1, n8s[C - 1])

        for c in range(C):
            x_rdmas[c].wait_recv()
            rows = pl.ds(c * R, R)
            tok_rows = pl.ds(tok0 + c * R, R)
            mine = (ids_vmem[tok_rows] // V_SHARD) == x
            out_ref[tok_rows] = jnp.where(mine, part_ref[rows], xrecv_ref[rows])
            y_rdmas[c].start()

        for c in range(C):
            y_rdmas[c].wait_recv()

        for c in range(C):
            x_rdmas[c].wait_send()
            y_rdmas[c].wait_send()

    return pl.pallas_call(
        body,
        out_shape=jax.ShapeDtypeStruct((T, D), jnp.bfloat16),
        in_specs=[
            pl.BlockSpec(memory_space=pltpu.SMEM),
            pl.BlockSpec(memory_space=pltpu.VMEM),
            pl.BlockSpec(memory_space=pltpu.HBM),
        ],
        out_specs=pl.BlockSpec(memory_space=pltpu.VMEM),
        scratch_shapes=[
            pltpu.VMEM((HALF + 1, D), jnp.float32),
            pltpu.VMEM((HALF, D), jnp.bfloat16),
            pltpu.VMEM((HALF, D), jnp.bfloat16),
            pltpu.SMEM((C * SEG,), jnp.int32),
            pltpu.SMEM((C * SEG,), jnp.int32),
            pltpu.SemaphoreType.DMA((C,)),
            pltpu.SemaphoreType.DMA((C,)),
            pltpu.SemaphoreType.DMA((C,)),
            pltpu.SemaphoreType.DMA((C,)),
            pltpu.SemaphoreType.DMA((C,)),
        ],
        compiler_params=pltpu.CompilerParams(collective_id=0),
    )(idx_loc, ids2d, E)


# device time: 36716 ns/iter; 1.4305x vs baseline; 1.2158x over previous
import jax
import jax.numpy as jnp
from jax import lax
from jax.experimental import pallas as pl
from jax.experimental.pallas import tpu as pltpu

T = 2048
D = 1024
V_SHARD = 16384
HALF = T // 2
C = 16
R = HALF // C
G = 8
SEG = R + G
LAG = 2


def kernel(ids, E):
    ids1d = ids.astype(jnp.int32)
    ids2d = ids1d.reshape(T, 1)

    my_x = lax.axis_index("x")
    my_y = lax.axis_index("y")

    seg_ids = lax.dynamic_slice(ids1d, (my_y * HALF,), (HALF,))
    idx_loc = seg_ids - my_x * V_SHARD

    def body(
        idx_smem,
        ids_vmem,
        E_hbm,
        out_ref,
        gbuf_ref,
        part_ref,
        xrecv_ref,
        src_l,
        dst_l,
        gsems,
        x_send_sems,
        x_recv_sems,
        y_send_sems,
        y_recv_sems,
    ):
        x = lax.axis_index("x")
        y = lax.axis_index("y")
        xnbr = (1 - x, y)
        ynbr = (x, 1 - y)

        barrier = pltpu.get_barrier_semaphore()
        for nbr in (xnbr, ynbr):
            pl.semaphore_signal(
                barrier, inc=1, device_id=nbr, device_id_type=pl.DeviceIdType.MESH
            )

        tok0 = y * HALF

        x_rdmas = []
        y_rdmas = []
        for c in range(C):
            rows = pl.ds(c * R, R)
            tok_rows = pl.ds(tok0 + c * R, R)
            x_rdmas.append(
                pltpu.make_async_remote_copy(
                    src_ref=part_ref.at[rows],
                    dst_ref=xrecv_ref.at[rows],
                    send_sem=x_send_sems.at[c],
                    recv_sem=x_recv_sems.at[c],
                    device_id=xnbr,
                    device_id_type=pl.DeviceIdType.MESH,
                )
            )
            y_rdmas.append(
                pltpu.make_async_remote_copy(
                    src_ref=out_ref.at[tok_rows],
                    dst_ref=out_ref.at[tok_rows],
                    send_sem=y_send_sems.at[c],
                    recv_sem=y_recv_sems.at[c],
                    device_id=ynbr,
                    device_id_type=pl.DeviceIdType.MESH,
                )
            )

        def scan_issue_chunk(c):
            base = c * SEG

            def scan_row(t, cnt):
                v = idx_smem[c * R + t]
                ok = jnp.logical_and(v >= 0, v < V_SHARD)
                src_l[base + cnt] = v
                dst_l[base + cnt] = c * R + t
                return cnt + ok.astype(jnp.int32)

            cnt = lax.fori_loop(0, R, scan_row, 0, unroll=4)
            for u in range(G):
                src_l[base + cnt + u] = 0
                dst_l[base + cnt + u] = HALF
            n8 = (cnt + G - 1) // G

            def body8(k, _):
                off = base + k * G
                for u in range(G):
                    pltpu.make_async_copy(
                        E_hbm.at[pl.ds(src_l[off + u], 1), :],
                        gbuf_ref.at[pl.ds(dst_l[off + u], 1), :],
                        gsems.at[c],
                    ).start()
                return 0

            lax.fori_loop(0, n8, body8, 0)
            return n8

        def flush_chunk(c, n8):
            def wait8(k, _):
                for _u in range(G):
                    pltpu.make_async_copy(
                        E_hbm.at[pl.ds(0, 1), :],
                        gbuf_ref.at[pl.ds(0, 1), :],
                        gsems.at[c],
                    ).wait()
                return 0

            lax.fori_loop(0, n8, wait8, 0)
            rows = pl.ds(c * R, R)
            part_ref[rows] = gbuf_ref[rows].astype(jnp.bfloat16)
            x_rdmas[c].start()

        def combine_chunk(c):
            x_rdmas[c].wait_recv()
            rows = pl.ds(c * R, R)
            tok_rows = pl.ds(tok0 + c * R, R)
            mine = (ids_vmem[tok_rows] // V_SHARD) == x
            out_ref[tok_rows] = jnp.where(mine, part_ref[rows], xrecv_ref[rows])
            y_rdmas[c].start()

        n8s = [None] * C
        n8s[0] = scan_issue_chunk(0)
        pl.semaphore_wait(barrier, 2)
        for c in range(1, C):
            n8s[c] = scan_issue_chunk(c)
            flush_chunk(c - 1, n8s[c - 1])
            if c - 1 >= LAG:
                combine_chunk(c - 1 - LAG)
        flush_chunk(C - 1, n8s[C - 1])
        for c in range(C - 1 - LAG, C):
            combine_chunk(c)

        for c in range(C):
            y_rdmas[c].wait_recv()

        for c in range(C):
            x_rdmas[c].wait_send()
            y_rdmas[c].wait_send()

    return pl.pallas_call(
        body,
        out_shape=jax.ShapeDtypeStruct((T, D), jnp.bfloat16),
        in_specs=[
            pl.BlockSpec(memory_space=pltpu.SMEM),
            pl.BlockSpec(memory_space=pltpu.VMEM),
            pl.BlockSpec(memory_space=pltpu.HBM),
        ],
        out_specs=pl.BlockSpec(memory_space=pltpu.VMEM),
        scratch_shapes=[
            pltpu.VMEM((HALF + 1, D), jnp.float32),
            pltpu.VMEM((HALF, D), jnp.bfloat16),
            pltpu.VMEM((HALF, D), jnp.bfloat16),
            pltpu.SMEM((C * SEG,), jnp.int32),
            pltpu.SMEM((C * SEG,), jnp.int32),
            pltpu.SemaphoreType.DMA((C,)),
            pltpu.SemaphoreType.DMA((C,)),
            pltpu.SemaphoreType.DMA((C,)),
            pltpu.SemaphoreType.DMA((C,)),
            pltpu.SemaphoreType.DMA((C,)),
        ],
        compiler_params=pltpu.CompilerParams(collective_id=0),
    )(idx_loc, ids2d, E)


# device time: 36691 ns/iter; 1.4314x vs baseline; 1.0007x over previous
import jax
import jax.numpy as jnp
from jax import lax
from jax.experimental import pallas as pl
from jax.experimental.pallas import tpu as pltpu

T = 2048
D = 1024
V_SHARD = 16384
HALF = T // 2
C = 16
R = HALF // C
G = 8
SEG = R + G
LAG = 3


def kernel(ids, E):
    ids1d = ids.astype(jnp.int32)
    ids2d = ids1d.reshape(T, 1)

    my_x = lax.axis_index("x")
    my_y = lax.axis_index("y")

    seg_ids = lax.dynamic_slice(ids1d, (my_y * HALF,), (HALF,))
    idx_loc = seg_ids - my_x * V_SHARD

    def body(
        idx_smem,
        ids_vmem,
        E_hbm,
        out_ref,
        gbuf_ref,
        part_ref,
        xrecv_ref,
        src_l,
        dst_l,
        gsems,
        x_send_sems,
        x_recv_sems,
        y_send_sems,
        y_recv_sems,
    ):
        x = lax.axis_index("x")
        y = lax.axis_index("y")
        xnbr = (1 - x, y)
        ynbr = (x, 1 - y)

        barrier = pltpu.get_barrier_semaphore()
        for nbr in (xnbr, ynbr):
            pl.semaphore_signal(
                barrier, inc=1, device_id=nbr, device_id_type=pl.DeviceIdType.MESH
            )

        tok0 = y * HALF

        x_rdmas = []
        y_rdmas = []
        for c in range(C):
            rows = pl.ds(c * R, R)
            tok_rows = pl.ds(tok0 + c * R, R)
            x_rdmas.append(
                pltpu.make_async_remote_copy(
                    src_ref=part_ref.at[rows],
                    dst_ref=xrecv_ref.at[rows],
                    send_sem=x_send_sems.at[c],
                    recv_sem=x_recv_sems.at[c],
                    device_id=xnbr,
                    device_id_type=pl.DeviceIdType.MESH,
                )
            )
            y_rdmas.append(
                pltpu.make_async_remote_copy(
                    src_ref=out_ref.at[tok_rows],
                    dst_ref=out_ref.at[tok_rows],
                    send_sem=y_send_sems.at[c],
                    recv_sem=y_recv_sems.at[c],
                    device_id=ynbr,
                    device_id_type=pl.DeviceIdType.MESH,
                )
            )

        def scan_issue_chunk(c):
            base = c * SEG

            def scan_row(t, cnt):
                v = idx_smem[c * R + t]
                ok = jnp.logical_and(v >= 0, v < V_SHARD)
                src_l[base + cnt] = v
                dst_l[base + cnt] = c * R + t
                return cnt + ok.astype(jnp.int32)

            cnt = lax.fori_loop(0, R, scan_row, 0, unroll=4)
            for u in range(G):
                src_l[base + cnt + u] = 0
                dst_l[base + cnt + u] = HALF
            n8 = (cnt + G - 1) // G

            def body8(k, _):
                off = base + k * G
                for u in range(G):
                    pltpu.make_async_copy(
                        E_hbm.at[pl.ds(src_l[off + u], 1), :],
                        gbuf_ref.at[pl.ds(dst_l[off + u], 1), :],
                        gsems.at[c],
                    ).start()
                return 0

            lax.fori_loop(0, n8, body8, 0)
            return n8

        def flush_chunk(c, n8):
            def wait8(k, _):
                for _u in range(G):
                    pltpu.make_async_copy(
                        E_hbm.at[pl.ds(0, 1), :],
                        gbuf_ref.at[pl.ds(0, 1), :],
                        gsems.at[c],
                    ).wait()
                return 0

            lax.fori_loop(0, n8, wait8, 0)
            rows = pl.ds(c * R, R)
            part_ref[rows] = gbuf_ref[rows].astype(jnp.bfloat16)
            x_rdmas[c].start()

        def combine_chunk(c):
            x_rdmas[c].wait_recv()
            rows = pl.ds(c * R, R)
            tok_rows = pl.ds(tok0 + c * R, R)
            mine = (ids_vmem[tok_rows] // V_SHARD) == x
            out_ref[tok_rows] = jnp.where(mine, part_ref[rows], xrecv_ref[rows])
            y_rdmas[c].start()

        n8s = [None] * C
        n8s[0] = scan_issue_chunk(0)
        pl.semaphore_wait(barrier, 2)
        for c in range(1, C):
            n8s[c] = scan_issue_chunk(c)
            flush_chunk(c - 1, n8s[c - 1])
            if c - 1 >= LAG:
                combine_chunk(c - 1 - LAG)
        flush_chunk(C - 1, n8s[C - 1])
        for c in range(C - 1 - LAG, C):
            combine_chunk(c)

        for c in range(C):
            y_rdmas[c].wait_recv()

        for c in range(C):
            x_rdmas[c].wait_send()
            y_rdmas[c].wait_send()

    return pl.pallas_call(
        body,
        out_shape=jax.ShapeDtypeStruct((T, D), jnp.bfloat16),
        in_specs=[
            pl.BlockSpec(memory_space=pltpu.SMEM),
            pl.BlockSpec(memory_space=pltpu.VMEM),
            pl.BlockSpec(memory_space=pltpu.HBM),
        ],
        out_specs=pl.BlockSpec(memory_space=pltpu.VMEM),
        scratch_shapes=[
            pltpu.VMEM((HALF + 1, D), jnp.float32),
            pltpu.VMEM((HALF, D), jnp.bfloat16),
            pltpu.VMEM((HALF, D), jnp.bfloat16),
            pltpu.SMEM((C * SEG,), jnp.int32),
            pltpu.SMEM((C * SEG,), jnp.int32),
            pltpu.SemaphoreType.DMA((C,)),
            pltpu.SemaphoreType.DMA((C,)),
            pltpu.SemaphoreType.DMA((C,)),
            pltpu.SemaphoreType.DMA((C,)),
            pltpu.SemaphoreType.DMA((C,)),
        ],
        compiler_params=pltpu.CompilerParams(collective_id=0),
    )(idx_loc, ids2d, E)
